# baseline (speedup 1.0000x reference)
import functools

import jax
import jax.numpy as jnp
from jax import lax
from jax.experimental import pallas as pl
from jax.experimental.pallas import tpu as pltpu
from jax.experimental.pallas import tpu_sc as plsc

NB = 8
N = 1024
FI = 8
TT = 12
FC = 64
NE = 32768
NL = FI * TT

_NC = 2
_NS = 16
_TILES = 16
_K = 16
_CH = 128
_SLICE = (N * N) // _TILES


def _sc_build_lhat(rows3, cols3):
  mesh = plsc.VectorSubcoreMesh(
      core_axis_name="c", subcore_axis_name="s",
      num_cores=_NC, num_subcores=_NS)

  @functools.partial(
      pl.kernel,
      out_type=jax.ShapeDtypeStruct((N * N,), jnp.float32),
      mesh=mesh,
      scratch_types=[
          pltpu.VMEM((_K, _CH), jnp.int32),
          pltpu.VMEM((_K, _CH), jnp.int32),
          pltpu.VMEM((_K, _CH), jnp.int32),
          pltpu.VMEM((_K, _CH), jnp.float32),
          pltpu.VMEM((2048,), jnp.float32),
          pltpu.VMEM((N,), jnp.float32),
          pltpu.VMEM((N,), jnp.float32),
          pltpu.VMEM_SHARED((N,), jnp.float32),
          pltpu.VMEM_SHARED((N * N,), jnp.float32),
      ],
      compiler_params=pltpu.CompilerParams(needs_layout_passes=False),
  )
  def built(rows_hbm, cols_hbm, out_hbm, rv, cv, fv, wv, zv, degv, disv,
            deg_sh, lhat_sh):
    cid = lax.axis_index("c")
    sid = lax.axis_index("s")
    on0 = cid == 0

    z16 = jnp.zeros((16,), jnp.float32)
    for i in range(2048 // 16):
      zv[pl.ds(i * 16, 16)] = z16

    @pl.when(on0)
    def _zero_and_load():
      for i in range(_SLICE // 2048):
        pltpu.sync_copy(zv, lhat_sh.at[pl.ds(sid * _SLICE + i * 2048, 2048)])

      @pl.when(sid == 0)
      def _zero_deg():
        pltpu.sync_copy(zv.at[pl.ds(0, N)], deg_sh)

      pltpu.sync_copy(rows_hbm.at[sid], rv)
      pltpu.sync_copy(cols_hbm.at[sid], cv)

    plsc.subcore_barrier()

    @pl.when(on0)
    def _deg_scatter():
      for j in range(_K):
        for l in range(_CH // 16):
          r = rv[j, pl.ds(l * 16, 16)]
          c = cv[j, pl.ds(l * 16, 16)]
          w = jnp.where(r != c, 1.0, 0.0).astype(jnp.float32)
          wv[j, pl.ds(l * 16, 16)] = w
          fv[j, pl.ds(l * 16, 16)] = r * N + c
      for j in range(_K):
        pltpu.sync_copy(wv.at[j], deg_sh.at[rv.at[j]], add=True)

    plsc.subcore_barrier()

    @pl.when(on0)
    def _edge_scatter():
      pltpu.sync_copy(deg_sh, degv)
      seeds = [1.0 / float((1.5 * (2.0 ** j)) ** 0.5) for j in range(16)]
      for i in range(N // 16):
        d = degv[pl.ds(i * 16, 16)]
        x = jnp.full((16,), seeds[0], jnp.float32)
        for j in range(1, 16):
          x = jnp.where(d >= float(2 ** j), seeds[j], x)
        for _ in range(4):
          x = x * (1.5 - 0.5 * d * x * x)
        disv[pl.ds(i * 16, 16)] = jnp.where(d > 0.5, x, 0.0)
      for j in range(_K):
        for l in range(_CH // 16):
          r = rv[j, pl.ds(l * 16, 16)]
          c = cv[j, pl.ds(l * 16, 16)]
          dr = plsc.load_gather(disv, [r])
          dc = plsc.load_gather(disv, [c])
          wv[j, pl.ds(l * 16, 16)] = jnp.where(r != c, -(dr * dc), 0.0)
      for j in range(_K):
        pltpu.sync_copy(wv.at[j], lhat_sh.at[fv.at[j]], add=True)

    plsc.subcore_barrier()

    @pl.when(on0)
    def _writeback():
      pltpu.sync_copy(lhat_sh.at[pl.ds(sid * _SLICE, _SLICE)],
                      out_hbm.at[pl.ds(sid * _SLICE, _SLICE)])

  return built(rows3, cols3)


def _et_body(x_ref, u1_ref, u2_ref, mu3_ref, eye96_ref, psplit_ref, pt_ref,
             ve_ref, be_ref, et_ref):
  x = x_ref[0]
  v = jnp.dot(u1_ref[...], x)
  trhs = jnp.dot(x, mu3_ref[...])
  g = jnp.dot(u2_ref[...], trhs)
  vd = eye96_ref[...] * v
  vft = jnp.dot(psplit_ref[...], jnp.dot(vd, pt_ref[...]))
  prod = jnp.dot(vft.T, g)
  emat = jnp.dot(ve_ref[...], jax.nn.sigmoid(prod + be_ref[...]))
  m = jnp.max(emat, axis=0, keepdims=True)
  e = jnp.exp(emat - m)
  et_ref[0] = e / jnp.sum(e, axis=0, keepdims=True)


def _main_body(x_ref, bd_ref, lhat_ref, bs_ref, vs_ref, m1w_ref, w2_ref,
               m3w_ref, wb_ref, wtjt_ref, wrt_ref, cb_ref, bt_ref, br_ref,
               lnw_ref, lnb_ref, out_ref):
  x = x_ref[0]
  xt = jnp.dot(x, bd_ref[0])
  slhs = jnp.dot(jnp.dot(xt, m1w_ref[...]), w2_ref[...])
  r12 = jnp.dot(xt, m3w_ref[...])
  p = lax.dot_general(slhs, r12, (((1,), (1,)), ((), ())),
                      preferred_element_type=jnp.float32)
  sg = jax.nn.sigmoid(p + bs_ref[...])
  smat = jnp.dot(vs_ref[...].astype(jnp.bfloat16), sg.astype(jnp.bfloat16),
                 preferred_element_type=jnp.float32)
  e = jnp.exp(smat)
  colsum = jnp.dot(jnp.ones((1, N), jnp.float32), e)
  s = e * (1.0 / colsum)

  ri = lax.broadcasted_iota(jnp.int32, (N, N), 0)
  ci = lax.broadcasted_iota(jnp.int32, (N, N), 1)
  dvec = jnp.sum(jnp.where(ri == ci, s, 0.0), axis=1, keepdims=True)

  y0 = x * dvec
  a1 = lhat_ref[...] * s
  y1 = jnp.dot(a1, y0)
  y2 = 2.0 * jnp.dot(lhat_ref[...], y1) - y0

  cb = cb_ref[...]
  xhat = []
  for t in range(TT):
    o = (jnp.dot(y0, wb_ref[0, t]) + jnp.dot(y1, wb_ref[1, t])
         + jnp.dot(y2, wb_ref[2, t]) + cb)
    xhat.append(jnp.maximum(o, 0.0))

  for t in range(TT):
    xh = jnp.dot(xhat[t], wtjt_ref[1])
    if t > 0:
      xh = xh + jnp.dot(xhat[t - 1], wtjt_ref[0])
    if t < TT - 1:
      xh = xh + jnp.dot(xhat[t + 1], wtjt_ref[2])
    xr = jnp.dot(x, wrt_ref[t])
    z = jnp.maximum(xh + bt_ref[...] + xr + br_ref[...], 0.0)
    ones_c = jnp.full((FC, 1), 1.0 / FC, jnp.float32)
    mu = jnp.dot(z, ones_c)
    zc = z - mu
    var = jnp.dot(zc * zc, ones_c)
    out_ref[0, t] = (zc * lax.rsqrt(var + 1e-5) * lnw_ref[...]
                     + lnb_ref[...])


def _et_call(x96, u1r, u2, mu3, eye96, psplit, pt, ve, be0, interpret=False):
  full = lambda shape: pl.BlockSpec(shape, lambda b: (0,) * len(shape))
  return pl.pallas_call(
      _et_body,
      grid=(NB,),
      in_specs=[
          pl.BlockSpec((1, N, NL), lambda b: (b, 0, 0)),
          full((1, N)), full((FI, N)), full((NL, TT)), full((NL, NL)),
          full((FI, NL)), full((NL, TT)), full((TT, TT)), full((TT, TT)),
      ],
      out_specs=pl.BlockSpec((1, TT, TT), lambda b: (b, 0, 0)),
      out_shape=jax.ShapeDtypeStruct((NB, TT, TT), jnp.float32),
      interpret=interpret,
  )(x96, u1r, u2, mu3, eye96, psplit, pt, ve, be0)


def _main_call(x96, bd, lhat, bs0, vs, m1w, w2, m3w, wb, wtjt, wrt, cb, bt,
               br, lnw, lnb, interpret=False):
  full = lambda shape: pl.BlockSpec(shape, lambda b: (0,) * len(shape))
  return pl.pallas_call(
      _main_body,
      grid=(NB,),
      in_specs=[
          pl.BlockSpec((1, N, NL), lambda b: (b, 0, 0)),
          pl.BlockSpec((1, NL, NL), lambda b: (b, 0, 0)),
          full((N, N)), full((N, N)), full((N, N)),
          full((NL, FI)), full((FI, TT)), full((NL, TT)),
          full((3, TT, NL, FC)), full((3, FC, FC)), full((TT, NL, FC)),
          full((1, FC)), full((1, FC)), full((1, FC)),
          full((1, FC)), full((1, FC)),
      ],
      out_specs=pl.BlockSpec((1, TT, N, FC), lambda b: (b, 0, 0, 0)),
      out_shape=jax.ShapeDtypeStruct((NB, TT, N, FC), jnp.float32),
      interpret=interpret,
  )(x96, bd, lhat, bs0, vs, m1w, w2, m3w, wb, wtjt, wrt, cb, bt, br, lnw, lnb)


def kernel(X, edge_index, U1, U2, U3, be, Ve, W1, W2, W3, bs, Vs, cheb_w,
           cheb_b, W_time, b_time, W_res, b_res, ln_w, ln_b):
  x96 = X.reshape(NB, N, NL)
  rows3 = edge_index[0].reshape(_TILES, _K, _CH)
  cols3 = edge_index[1].reshape(_TILES, _K, _CH)

  lhat = _sc_build_lhat(rows3, cols3).reshape(N, N)

  eye8 = jnp.eye(FI, dtype=jnp.float32)
  eye12 = jnp.eye(TT, dtype=jnp.float32)
  mu3 = (U3[:, None, None] * eye12[None]).reshape(NL, TT)
  psplit = jnp.repeat(eye8, TT, axis=1)
  pt = jnp.tile(eye12, (FI, 1))
  eye96 = jnp.eye(NL, dtype=jnp.float32)

  et = _et_call(x96, U1.reshape(1, N), U2, mu3, eye96, psplit, pt, Ve, be[0])

  bd = (eye8[None, :, None, :, None]
        * et[:, None, :, None, :]).reshape(NB, NL, NL)

  m1w = jnp.kron(eye8, W1[:, None])
  m3w = jnp.kron(W3[:, None], eye12)
  wb = (cheb_w[:, None, :, None, :]
        * eye12[None, :, None, :, None]).reshape(3, TT, NL, FC)
  wtjt = jnp.transpose(W_time[:, :, 0, :], (2, 1, 0))
  wrt = (W_res[:, :, 0, 0].T[None, :, None, :]
         * eye12[:, None, :, None]).reshape(TT, NL, FC)

  out = _main_call(x96, bd, lhat, bs[0], Vs, m1w, W2, m3w, wb, wtjt, wrt,
                   cheb_b.reshape(1, FC), b_time.reshape(1, FC),
                   b_res.reshape(1, FC), ln_w.reshape(1, FC),
                   ln_b.reshape(1, FC))
  return jnp.transpose(out, (0, 2, 3, 1))


# speedup vs baseline: 1.2707x; 1.2707x over previous
import functools

import jax
import jax.numpy as jnp
from jax import lax
from jax.experimental import pallas as pl
from jax.experimental.pallas import tpu as pltpu
from jax.experimental.pallas import tpu_sc as plsc

NB = 8
N = 1024
FI = 8
TT = 12
FC = 64
NE = 32768
NL = FI * TT

_NC = 2
_NS = 16
_TILES = 16
_K = 16
_CH = 128
_SLICE = (N * N) // _TILES


def _sc_build_lhat(rows3, cols3):
  mesh = plsc.VectorSubcoreMesh(
      core_axis_name="c", subcore_axis_name="s",
      num_cores=_NC, num_subcores=_NS)

  @functools.partial(
      pl.kernel,
      out_type=jax.ShapeDtypeStruct((N * N,), jnp.float32),
      mesh=mesh,
      scratch_types=[
          pltpu.VMEM((_K, _CH), jnp.int32),
          pltpu.VMEM((_K, _CH), jnp.int32),
          pltpu.VMEM((_K, _CH), jnp.int32),
          pltpu.VMEM((_K, _CH), jnp.float32),
          pltpu.VMEM((2048,), jnp.float32),
          pltpu.VMEM((N,), jnp.float32),
          pltpu.VMEM((N,), jnp.float32),
          pltpu.VMEM_SHARED((N,), jnp.float32),
          pltpu.VMEM_SHARED((N * N,), jnp.float32),
      ],
      compiler_params=pltpu.CompilerParams(needs_layout_passes=False),
  )
  def built(rows_hbm, cols_hbm, out_hbm, rv, cv, fv, wv, zv, degv, disv,
            deg_sh, lhat_sh):
    cid = lax.axis_index("c")
    sid = lax.axis_index("s")
    on0 = cid == 0

    z16 = jnp.zeros((16,), jnp.float32)
    for i in range(2048 // 16):
      zv[pl.ds(i * 16, 16)] = z16

    @pl.when(on0)
    def _zero_and_load():
      for i in range(_SLICE // 2048):
        pltpu.sync_copy(zv, lhat_sh.at[pl.ds(sid * _SLICE + i * 2048, 2048)])

      @pl.when(sid == 0)
      def _zero_deg():
        pltpu.sync_copy(zv.at[pl.ds(0, N)], deg_sh)

      pltpu.sync_copy(rows_hbm.at[sid], rv)
      pltpu.sync_copy(cols_hbm.at[sid], cv)

    plsc.subcore_barrier()

    @pl.when(on0)
    def _deg_scatter():
      for j in range(_K):
        for l in range(_CH // 16):
          r = rv[j, pl.ds(l * 16, 16)]
          c = cv[j, pl.ds(l * 16, 16)]
          w = jnp.where(r != c, 1.0, 0.0).astype(jnp.float32)
          wv[j, pl.ds(l * 16, 16)] = w
          fv[j, pl.ds(l * 16, 16)] = r * N + c
      for j in range(_K):
        pltpu.sync_copy(wv.at[j], deg_sh.at[rv.at[j]], add=True)

    plsc.subcore_barrier()

    @pl.when(on0)
    def _edge_scatter():
      pltpu.sync_copy(deg_sh, degv)
      seeds = [1.0 / float((1.5 * (2.0 ** j)) ** 0.5) for j in range(16)]
      for i in range(N // 16):
        d = degv[pl.ds(i * 16, 16)]
        x = jnp.full((16,), seeds[0], jnp.float32)
        for j in range(1, 16):
          x = jnp.where(d >= float(2 ** j), seeds[j], x)
        for _ in range(4):
          x = x * (1.5 - 0.5 * d * x * x)
        disv[pl.ds(i * 16, 16)] = jnp.where(d > 0.5, x, 0.0)
      for j in range(_K):
        for l in range(_CH // 16):
          r = rv[j, pl.ds(l * 16, 16)]
          c = cv[j, pl.ds(l * 16, 16)]
          dr = plsc.load_gather(disv, [r])
          dc = plsc.load_gather(disv, [c])
          wv[j, pl.ds(l * 16, 16)] = jnp.where(r != c, -(dr * dc), 0.0)
      for j in range(_K):
        pltpu.sync_copy(wv.at[j], lhat_sh.at[fv.at[j]], add=True)

    plsc.subcore_barrier()

    @pl.when(on0)
    def _writeback():
      pltpu.sync_copy(lhat_sh.at[pl.ds(sid * _SLICE, _SLICE)],
                      out_hbm.at[pl.ds(sid * _SLICE, _SLICE)])

  return built(rows3, cols3)


def _et_body(x_ref, u1_ref, u2_ref, mu3_ref, eye96_ref, psplit_ref, pt_ref,
             ve_ref, be_ref, et_ref):
  x = x_ref[0]
  v = jnp.dot(u1_ref[...], x)
  trhs = jnp.dot(x, mu3_ref[...])
  g = jnp.dot(u2_ref[...], trhs)
  vd = eye96_ref[...] * v
  vft = jnp.dot(psplit_ref[...], jnp.dot(vd, pt_ref[...]))
  prod = jnp.dot(vft.T, g)
  emat = jnp.dot(ve_ref[...], jax.nn.sigmoid(prod + be_ref[...]))
  m = jnp.max(emat, axis=0, keepdims=True)
  e = jnp.exp(emat - m)
  et_ref[0] = e / jnp.sum(e, axis=0, keepdims=True)


def _main_body(x_ref, bd_ref, lhat_ref, bs_ref, vs_ref, m1w_ref, w2_ref,
               m3w_ref, wb_ref, wtjt_ref, wrt_ref, cb_ref, bt_ref, br_ref,
               lnw_ref, lnb_ref, out_ref):
  x = x_ref[0]
  xt = jnp.dot(x, bd_ref[0])
  slhs = jnp.dot(jnp.dot(xt, m1w_ref[...]), w2_ref[...])
  r12 = jnp.dot(xt, m3w_ref[...])
  p = lax.dot_general(slhs, r12, (((1,), (1,)), ((), ())),
                      preferred_element_type=jnp.float32)
  sg = jax.nn.sigmoid(p + bs_ref[...])
  smat = jnp.dot(vs_ref[...].astype(jnp.bfloat16), sg.astype(jnp.bfloat16),
                 preferred_element_type=jnp.float32)
  m = jnp.max(smat, axis=0, keepdims=True)
  e = jnp.exp(smat - m)
  s = e / jnp.sum(e, axis=0, keepdims=True)

  ri = lax.broadcasted_iota(jnp.int32, (N, N), 0)
  ci = lax.broadcasted_iota(jnp.int32, (N, N), 1)
  dvec = jnp.sum(jnp.where(ri == ci, s, 0.0), axis=1, keepdims=True)

  y0 = x * dvec
  a1 = lhat_ref[...] * s
  y1 = jnp.dot(a1, y0)
  y2 = 2.0 * jnp.dot(lhat_ref[...], y1) - y0

  cb = cb_ref[...]
  xhat = []
  for t in range(TT):
    o = (jnp.dot(y0, wb_ref[0, t]) + jnp.dot(y1, wb_ref[1, t])
         + jnp.dot(y2, wb_ref[2, t]) + cb)
    xhat.append(jnp.maximum(o, 0.0))

  for t in range(TT):
    xh = jnp.dot(xhat[t], wtjt_ref[1])
    if t > 0:
      xh = xh + jnp.dot(xhat[t - 1], wtjt_ref[0])
    if t < TT - 1:
      xh = xh + jnp.dot(xhat[t + 1], wtjt_ref[2])
    xr = jnp.dot(x, wrt_ref[t])
    z = jnp.maximum(xh + bt_ref[...] + xr + br_ref[...], 0.0)
    mu = jnp.mean(z, axis=1, keepdims=True)
    zc = z - mu
    var = jnp.mean(zc * zc, axis=1, keepdims=True)
    out_ref[0, t] = (zc * lax.rsqrt(var + 1e-5) * lnw_ref[...]
                     + lnb_ref[...])


def _et_call(x96, u1r, u2, mu3, eye96, psplit, pt, ve, be0, interpret=False):
  full = lambda shape: pl.BlockSpec(shape, lambda b: (0,) * len(shape))
  return pl.pallas_call(
      _et_body,
      grid=(NB,),
      in_specs=[
          pl.BlockSpec((1, N, NL), lambda b: (b, 0, 0)),
          full((1, N)), full((FI, N)), full((NL, TT)), full((NL, NL)),
          full((FI, NL)), full((NL, TT)), full((TT, TT)), full((TT, TT)),
      ],
      out_specs=pl.BlockSpec((1, TT, TT), lambda b: (b, 0, 0)),
      out_shape=jax.ShapeDtypeStruct((NB, TT, TT), jnp.float32),
      interpret=interpret,
  )(x96, u1r, u2, mu3, eye96, psplit, pt, ve, be0)


def _main_call(x96, bd, lhat, bs0, vs, m1w, w2, m3w, wb, wtjt, wrt, cb, bt,
               br, lnw, lnb, interpret=False):
  full = lambda shape: pl.BlockSpec(shape, lambda b: (0,) * len(shape))
  return pl.pallas_call(
      _main_body,
      grid=(NB,),
      in_specs=[
          pl.BlockSpec((1, N, NL), lambda b: (b, 0, 0)),
          pl.BlockSpec((1, NL, NL), lambda b: (b, 0, 0)),
          full((N, N)), full((N, N)), full((N, N)),
          full((NL, FI)), full((FI, TT)), full((NL, TT)),
          full((3, TT, NL, FC)), full((3, FC, FC)), full((TT, NL, FC)),
          full((1, FC)), full((1, FC)), full((1, FC)),
          full((1, FC)), full((1, FC)),
      ],
      out_specs=pl.BlockSpec((1, TT, N, FC), lambda b: (b, 0, 0, 0)),
      out_shape=jax.ShapeDtypeStruct((NB, TT, N, FC), jnp.float32),
      interpret=interpret,
  )(x96, bd, lhat, bs0, vs, m1w, w2, m3w, wb, wtjt, wrt, cb, bt, br, lnw, lnb)


def kernel(X, edge_index, U1, U2, U3, be, Ve, W1, W2, W3, bs, Vs, cheb_w,
           cheb_b, W_time, b_time, W_res, b_res, ln_w, ln_b):
  x96 = X.reshape(NB, N, NL)
  rows3 = edge_index[0].reshape(_TILES, _K, _CH)
  cols3 = edge_index[1].reshape(_TILES, _K, _CH)

  lhat = _sc_build_lhat(rows3, cols3).reshape(N, N)

  eye8 = jnp.eye(FI, dtype=jnp.float32)
  eye12 = jnp.eye(TT, dtype=jnp.float32)
  mu3 = (U3[:, None, None] * eye12[None]).reshape(NL, TT)
  psplit = jnp.repeat(eye8, TT, axis=1)
  pt = jnp.tile(eye12, (FI, 1))
  eye96 = jnp.eye(NL, dtype=jnp.float32)

  et = _et_call(x96, U1.reshape(1, N), U2, mu3, eye96, psplit, pt, Ve, be[0])

  bd = (eye8[None, :, None, :, None]
        * et[:, None, :, None, :]).reshape(NB, NL, NL)

  m1w = jnp.kron(eye8, W1[:, None])
  m3w = jnp.kron(W3[:, None], eye12)
  wb = (cheb_w[:, None, :, None, :]
        * eye12[None, :, None, :, None]).reshape(3, TT, NL, FC)
  wtjt = jnp.transpose(W_time[:, :, 0, :], (2, 1, 0))
  wrt = (W_res[:, :, 0, 0].T[None, :, None, :]
         * eye12[:, None, :, None]).reshape(TT, NL, FC)

  out = _main_call(x96, bd, lhat, bs[0], Vs, m1w, W2, m3w, wb, wtjt, wrt,
                   cheb_b.reshape(1, FC), b_time.reshape(1, FC),
                   b_res.reshape(1, FC), ln_w.reshape(1, FC),
                   ln_b.reshape(1, FC))
  return jnp.transpose(out, (0, 2, 3, 1))
